# Initial kernel scaffold; baseline (speedup 1.0000x reference)
import jax
import jax.numpy as jnp
from jax.experimental import pallas as pl


def kernel(candidate_embs, ffnn_W0, ffnn_b0, ffnn_W1, ffnn_b1, ffnn_Wout, ffnn_bout, gcn_fw_W, gcn_fw_b, gcn_bw_W, gcn_bw_b, lin1_W, lin1_b):
    raise NotImplementedError("write your pallas kernel here")


# speedup vs baseline: 1.0250x; 1.0250x over previous
import functools

import jax
import jax.numpy as jnp
from jax.experimental import pallas as pl
from jax.experimental.pallas import tpu as pltpu

N = 256
D = 256
H = 512
NREL = 3
NLAYERS = 2

BI = 8


def _probs_kernel(x_ref, xi_ref, w0_ref, b0_ref, w1_ref, b1_ref,
                  wout_ref, bout_ref, out_ref, xa_ref, xb_ref):
    i = pl.program_id(0)

    @pl.when(i == 0)
    def _():
        x = x_ref[...]
        xa_ref[...] = jnp.dot(x, w0_ref[:D, :],
                              preferred_element_type=jnp.float32)
        xb_ref[...] = jnp.dot(x, w0_ref[D:2 * D, :],
                              preferred_element_type=jnp.float32)

    x = x_ref[...]
    xi = xi_ref[...]
    p = (xi[:, None, :] * x[None, :, :]).reshape(BI * N, D)
    h = jnp.dot(p, w0_ref[2 * D:, :], preferred_element_type=jnp.float32)
    h = h.reshape(BI, N, H)
    h = h + xa_ref[...][None, :, :]
    h = h + xb_ref[pl.ds(i * BI, BI), :][:, None, :]
    h = jax.nn.relu(h + b0_ref[...][None, None, :]).reshape(BI * N, H)
    h = jax.nn.relu(jnp.dot(h, w1_ref[...],
                            preferred_element_type=jnp.float32)
                    + b1_ref[...][None, :])
    scores = jnp.dot(h, wout_ref[...],
                     preferred_element_type=jnp.float32) + bout_ref[...][None, :]
    cols = [scores[:, r] for r in range(NREL)]
    m = functools.reduce(jnp.maximum, cols)
    exps = [jnp.exp(c - m) for c in cols]
    denom = functools.reduce(jnp.add, exps)
    row_ids = i * BI + jax.lax.broadcasted_iota(jnp.int32, (BI, N), 0)
    col_ids = jax.lax.broadcasted_iota(jnp.int32, (BI, N), 1)
    mask = jnp.where(row_ids == col_ids, 0.0, 1.0)
    for r in range(NREL):
        out_ref[r] = (exps[r] / denom).reshape(BI, N) * mask


def _bigcn_kernel(probs_ref, x_ref, fww_ref, fwb_ref, bww_ref, bwb_ref,
                  l1w_ref, l1b_ref, out_ref):
    out = x_ref[...]
    for l in range(NLAYERS):
        rel_sum = jnp.zeros((N, D), dtype=jnp.float32)
        for r in range(NREL):
            a = probs_ref[r]
            fw = jnp.dot(a, jnp.dot(out, fww_ref[l, r],
                                    preferred_element_type=jnp.float32),
                         preferred_element_type=jnp.float32) + fwb_ref[l, r][None, :]
            bw = jnp.dot(a.T, jnp.dot(out, bww_ref[l, r],
                                      preferred_element_type=jnp.float32),
                         preferred_element_type=jnp.float32) + bwb_ref[l, r][None, :]
            rel_sum = rel_sum + jnp.concatenate([bw, fw], axis=-1)
        out = jnp.dot(jax.nn.relu(rel_sum), l1w_ref[l],
                      preferred_element_type=jnp.float32) + l1b_ref[l][None, :] + out
    out_ref[...] = out


def kernel(candidate_embs, ffnn_W0, ffnn_b0, ffnn_W1, ffnn_b1, ffnn_Wout,
           ffnn_bout, gcn_fw_W, gcn_fw_b, gcn_bw_W, gcn_bw_b, lin1_W, lin1_b):
    grid = N // BI
    full = lambda *shape: pl.BlockSpec(shape, lambda i: (0,) * len(shape))
    probs = pl.pallas_call(
        _probs_kernel,
        grid=(grid,),
        in_specs=[
            full(N, D),
            pl.BlockSpec((BI, D), lambda i: (i, 0)),
            full(3 * D, H),
            full(H),
            full(H, H),
            full(H),
            full(H, NREL),
            full(NREL),
        ],
        out_specs=pl.BlockSpec((NREL, BI, N), lambda i: (0, i, 0)),
        out_shape=jax.ShapeDtypeStruct((NREL, N, N), jnp.float32),
        scratch_shapes=[
            pltpu.VMEM((N, H), jnp.float32),
            pltpu.VMEM((N, H), jnp.float32),
        ],
    )(candidate_embs, candidate_embs, ffnn_W0, ffnn_b0, ffnn_W1, ffnn_b1,
      ffnn_Wout, ffnn_bout)

    out = pl.pallas_call(
        _bigcn_kernel,
        out_shape=jax.ShapeDtypeStruct((N, D), jnp.float32),
    )(probs, candidate_embs, gcn_fw_W, gcn_fw_b, gcn_bw_W, gcn_bw_b,
      lin1_W, lin1_b)
    return out
